# baseline (speedup 1.0000x reference)
import jax
import jax.numpy as jnp
from jax import lax
from jax.experimental import pallas as pl
from jax.experimental.pallas import tpu as pltpu
from jax.experimental.pallas import tpu_sc as plsc

N_CAT = 26
VOCAB = 100000
EMB = 64
N_NUM = 13
BATCH = 16384

NC = 2
NS = 16
NW = NC * NS
B_PER_W = BATCH // NW
SUB = 128
NSUB = B_PER_W // SUB


def _sc_gather_body(pair_idx, tables_pair, cat_out, idx_v, rows_v, sem):
    wid = lax.axis_index("s") * NC + lax.axis_index("c")
    base = wid * B_PER_W

    @pl.loop(0, N_CAT)
    def _field(i):
        pltpu.sync_copy(pair_idx.at[i, pl.ds(base, B_PER_W)], idx_v)
        cps = [
            pltpu.async_copy(
                tables_pair.at[idx_v.at[pl.ds(j * SUB, SUB)]], rows_v.at[j], sem
            )
            for j in range(NSUB)
        ]
        for c in cps:
            c.wait()
        for j in range(NSUB):
            pltpu.sync_copy(rows_v.at[j], cat_out.at[i, pl.ds(base + j * SUB, SUB)])


_sc_gather = pl.kernel(
    _sc_gather_body,
    out_type=jax.ShapeDtypeStruct((N_CAT, BATCH, 2 * EMB), jnp.float32),
    mesh=plsc.VectorSubcoreMesh(
        core_axis_name="c", subcore_axis_name="s", num_cores=NC, num_subcores=NS
    ),
    scratch_types=[
        pltpu.VMEM((B_PER_W,), jnp.int32),
        pltpu.VMEM((NSUB, SUB, 2 * EMB), jnp.float32),
        pltpu.SemaphoreType.DMA,
    ],
    compiler_params=pltpu.CompilerParams(use_tc_tiling_on_sc=True),
)

BB = 1024


def _tc_proj_body(cat_ref, par_ref, xn_ref, wn_ref, bn_ref, wfT_ref, bf_ref, out_ref):
    num = jnp.dot(xn_ref[...], wn_ref[...], preferred_element_type=jnp.float32)
    num = num + jnp.sum(bn_ref[...], axis=0, keepdims=True)
    acc = jnp.dot(num, wfT_ref[N_CAT * EMB :, :], preferred_element_type=jnp.float32)
    for i in range(N_CAT):
        lo = cat_ref[i, :, :EMB]
        hi = cat_ref[i, :, EMB:]
        par = par_ref[:, i : i + 1] > 0.5
        sel = jnp.where(par, hi, lo)
        acc = acc + jnp.dot(
            sel, wfT_ref[i * EMB : (i + 1) * EMB, :],
            preferred_element_type=jnp.float32,
        )
    out_ref[...] = acc + bf_ref[...]


_tc_proj = pl.pallas_call(
    _tc_proj_body,
    grid=(BATCH // BB,),
    in_specs=[
        pl.BlockSpec((N_CAT, BB, 2 * EMB), lambda b: (0, b, 0)),
        pl.BlockSpec((BB, N_CAT), lambda b: (b, 0)),
        pl.BlockSpec((BB, N_NUM), lambda b: (b, 0)),
        pl.BlockSpec((N_NUM, EMB), lambda b: (0, 0)),
        pl.BlockSpec((N_NUM, EMB), lambda b: (0, 0)),
        pl.BlockSpec((N_CAT * EMB + EMB, EMB), lambda b: (0, 0)),
        pl.BlockSpec((1, EMB), lambda b: (0, 0)),
    ],
    out_specs=pl.BlockSpec((BB, EMB), lambda b: (b, 0)),
    out_shape=jax.ShapeDtypeStruct((BATCH, EMB), jnp.float32),
)


def kernel(X_cat, X_num, tables, W_num, b_num, W_final, b_final):
    offs = (jnp.arange(N_CAT, dtype=jnp.int32) * VOCAB)[:, None]
    flat_idx = X_cat.T + offs
    pair_idx = flat_idx >> 1
    par = (X_cat & 1).astype(jnp.float32)
    tables_pair = tables.reshape(N_CAT * VOCAB // 2, 2 * EMB)
    cat = _sc_gather(pair_idx, tables_pair)
    return _tc_proj(
        cat, par, X_num, W_num, b_num, W_final.T, b_final.reshape(1, EMB)
    )


# speedup vs baseline: 2.1234x; 2.1234x over previous
import jax
import jax.numpy as jnp
from jax import lax
from jax.experimental import pallas as pl
from jax.experimental.pallas import tpu as pltpu
from jax.experimental.pallas import tpu_sc as plsc

N_CAT = 26
VOCAB = 100000
EMB = 64
N_NUM = 13
BATCH = 16384

NC = 2
NS = 16
NW = NC * NS
E_PER_W = EMB // NW
OUT_CHUNK = 8192
L = 16


def _sc_gather_body(xcat_t, tables_t, cat_t, idx_v, row_v, out_v, sem):
    wid = lax.axis_index("s") * NC + lax.axis_index("c")

    @pl.loop(0, N_CAT)
    def _field(i):
        pltpu.sync_copy(xcat_t.at[i], idx_v)
        for de in range(E_PER_W):
            e = wid * E_PER_W + de
            pltpu.async_copy(tables_t.at[i, e], row_v, sem).wait()

            @pl.loop(0, BATCH // OUT_CHUNK)
            def _chunk(c):
                @pl.loop(0, OUT_CHUNK // L, unroll=8)
                def _vec(k):
                    idxv = idx_v[pl.ds(c * OUT_CHUNK + k * L, L)]
                    out_v[pl.ds(k * L, L)] = plsc.load_gather(row_v, [idxv])

                pltpu.sync_copy(out_v, cat_t.at[i, e, pl.ds(c * OUT_CHUNK, OUT_CHUNK)])


_sc_gather = pl.kernel(
    _sc_gather_body,
    out_type=jax.ShapeDtypeStruct((N_CAT, EMB, BATCH), jnp.float32),
    mesh=plsc.VectorSubcoreMesh(
        core_axis_name="c", subcore_axis_name="s", num_cores=NC, num_subcores=NS
    ),
    scratch_types=[
        pltpu.VMEM((BATCH,), jnp.int32),
        pltpu.VMEM((VOCAB,), jnp.float32),
        pltpu.VMEM((OUT_CHUNK,), jnp.float32),
        pltpu.SemaphoreType.DMA,
    ],
    compiler_params=pltpu.CompilerParams(
        use_tc_tiling_on_sc=True, needs_layout_passes=False
    ),
)

BB = 2048


def _tc_proj_body(cat_ref, xn_ref, wn_ref, bn_ref, wfT_ref, bf_ref, out_ref):
    num = jnp.dot(xn_ref[...], wn_ref[...], preferred_element_type=jnp.float32)
    num = num + jnp.sum(bn_ref[...], axis=0, keepdims=True)
    acc = jnp.dot(num, wfT_ref[N_CAT * EMB :, :], preferred_element_type=jnp.float32)
    for i in range(N_CAT):
        acc = acc + lax.dot_general(
            cat_ref[i], wfT_ref[i * EMB : (i + 1) * EMB, :],
            dimension_numbers=(((0,), (0,)), ((), ())),
            preferred_element_type=jnp.float32,
        )
    out_ref[...] = acc + bf_ref[...]


_tc_proj = pl.pallas_call(
    _tc_proj_body,
    grid=(BATCH // BB,),
    in_specs=[
        pl.BlockSpec((N_CAT, EMB, BB), lambda b: (0, 0, b)),
        pl.BlockSpec((BB, N_NUM), lambda b: (b, 0)),
        pl.BlockSpec((N_NUM, EMB), lambda b: (0, 0)),
        pl.BlockSpec((N_NUM, EMB), lambda b: (0, 0)),
        pl.BlockSpec((N_CAT * EMB + EMB, EMB), lambda b: (0, 0)),
        pl.BlockSpec((1, EMB), lambda b: (0, 0)),
    ],
    out_specs=pl.BlockSpec((BB, EMB), lambda b: (b, 0)),
    out_shape=jax.ShapeDtypeStruct((BATCH, EMB), jnp.float32),
)


def kernel(X_cat, X_num, tables, W_num, b_num, W_final, b_final):
    tables_t = tables.transpose(0, 2, 1)
    xcat_t = X_cat.T
    cat_t = _sc_gather(xcat_t, tables_t)
    return _tc_proj(
        cat_t, X_num, W_num, b_num, W_final.T, b_final.reshape(1, EMB)
    )


# speedup vs baseline: 2.1324x; 1.0042x over previous
import jax
import jax.numpy as jnp
from jax import lax
from jax.experimental import pallas as pl
from jax.experimental.pallas import tpu as pltpu
from jax.experimental.pallas import tpu_sc as plsc

N_CAT = 26
VOCAB = 100000
EMB = 64
N_NUM = 13
BATCH = 16384

NC = 2
NS = 16
NW = NC * NS
E_PER_W = EMB // NW
OUT_CHUNK = 4096
L = 16


NCHUNK = BATCH // OUT_CHUNK


def _sc_gather_body(xcat_t, tables_t, cat_t, idx_v, row_v, out_v, sem_row, sem_out):
    wid = lax.axis_index("s") * NC + lax.axis_index("c")

    @pl.loop(0, N_CAT)
    def _field(i):
        pltpu.sync_copy(xcat_t.at[i], idx_v)
        for de in range(E_PER_W):
            e = wid * E_PER_W + de
            pltpu.async_copy(tables_t.at[i, e], row_v, sem_row).wait()

            pending = [None, None]
            for c in range(NCHUNK):
                b = c % 2
                if pending[b] is not None:
                    pending[b].wait()
                    pending[b] = None

                @pl.loop(0, OUT_CHUNK // L, unroll=8)
                def _vec(k):
                    idxv = idx_v[pl.ds(c * OUT_CHUNK + k * L, L)]
                    out_v[b, pl.ds(k * L, L)] = plsc.load_gather(row_v, [idxv])

                pending[b] = pltpu.async_copy(
                    out_v.at[b],
                    cat_t.at[i, e, pl.ds(c * OUT_CHUNK, OUT_CHUNK)],
                    sem_out,
                )
            for b in range(2):
                if pending[b] is not None:
                    pending[b].wait()


_sc_gather = pl.kernel(
    _sc_gather_body,
    out_type=jax.ShapeDtypeStruct((N_CAT, EMB, BATCH), jnp.float32),
    mesh=plsc.VectorSubcoreMesh(
        core_axis_name="c", subcore_axis_name="s", num_cores=NC, num_subcores=NS
    ),
    scratch_types=[
        pltpu.VMEM((BATCH,), jnp.int32),
        pltpu.VMEM((VOCAB,), jnp.float32),
        pltpu.VMEM((2, OUT_CHUNK), jnp.float32),
        pltpu.SemaphoreType.DMA,
        pltpu.SemaphoreType.DMA,
    ],
    compiler_params=pltpu.CompilerParams(
        use_tc_tiling_on_sc=True, needs_layout_passes=False
    ),
)

BB = 2048


def _tc_proj_body(cat_ref, xn_ref, wn_ref, bn_ref, wfT_ref, bf_ref, out_ref):
    num = jnp.dot(xn_ref[...], wn_ref[...], preferred_element_type=jnp.float32)
    num = num + jnp.sum(bn_ref[...], axis=0, keepdims=True)
    acc = jnp.dot(num, wfT_ref[N_CAT * EMB :, :], preferred_element_type=jnp.float32)
    for i in range(N_CAT):
        acc = acc + lax.dot_general(
            cat_ref[i], wfT_ref[i * EMB : (i + 1) * EMB, :],
            dimension_numbers=(((0,), (0,)), ((), ())),
            preferred_element_type=jnp.float32,
        )
    out_ref[...] = acc + bf_ref[...]


_tc_proj = pl.pallas_call(
    _tc_proj_body,
    grid=(BATCH // BB,),
    in_specs=[
        pl.BlockSpec((N_CAT, EMB, BB), lambda b: (0, 0, b)),
        pl.BlockSpec((BB, N_NUM), lambda b: (b, 0)),
        pl.BlockSpec((N_NUM, EMB), lambda b: (0, 0)),
        pl.BlockSpec((N_NUM, EMB), lambda b: (0, 0)),
        pl.BlockSpec((N_CAT * EMB + EMB, EMB), lambda b: (0, 0)),
        pl.BlockSpec((1, EMB), lambda b: (0, 0)),
    ],
    out_specs=pl.BlockSpec((BB, EMB), lambda b: (b, 0)),
    out_shape=jax.ShapeDtypeStruct((BATCH, EMB), jnp.float32),
)


def kernel(X_cat, X_num, tables, W_num, b_num, W_final, b_final):
    tables_t = tables.transpose(0, 2, 1)
    xcat_t = X_cat.T
    cat_t = _sc_gather(xcat_t, tables_t)
    return _tc_proj(
        cat_t, X_num, W_num, b_num, W_final.T, b_final.reshape(1, EMB)
    )
